# Initial kernel scaffold; baseline (speedup 1.0000x reference)
import functools

import jax
import jax.numpy as jnp
from jax import lax
from jax.experimental import pallas as pl
from jax.experimental.pallas import tpu as pltpu
from jax.experimental.pallas import tpu_sc as plsc

N = 10000
NROWS = 10016
RPS = NROWS // 16
E0 = 320000
E = E0 + N
NC, NS = 2, 16
NW = NC * NS
CH = 128
TE = 10368
EPAD = TE * NW
NCHUNK = TE // CH
D1, DP1 = 128, 144
D2, DP2 = 64, 80
NEG_SLOPE = 0.2


def _prep1_body(x_ref, w_ref, as_ref, ad_ref, htab_ref, an_ref):
    i = pl.program_id(0)
    blk = htab_ref.shape[0]
    h = jnp.dot(x_ref[...], w_ref[...], preferred_element_type=jnp.float32)
    rows = i * blk + lax.broadcasted_iota(jnp.int32, (blk, 1), 0)
    mask = rows < N
    h = jnp.where(mask, h, 0.0)
    htab_ref[:, :D1] = h
    htab_ref[:, D1:D1 + 1] = jnp.where(mask, 1.0, 0.0)
    htab_ref[:, D1 + 1:] = jnp.zeros((blk, DP1 - D1 - 1), jnp.float32)
    an_ref[:, 0:1] = jnp.sum(h * as_ref[...], axis=1, keepdims=True)
    an_ref[:, 1:2] = jnp.sum(h * ad_ref[...], axis=1, keepdims=True)


def _prep1(x_pad, W1, att_src1, att_dst1):
    blk = 2504
    return pl.pallas_call(
        _prep1_body,
        grid=(NROWS // blk,),
        in_specs=[
            pl.BlockSpec((blk, D1), lambda i: (i, 0)),
            pl.BlockSpec((D1, D1), lambda i: (0, 0)),
            pl.BlockSpec((1, D1), lambda i: (0, 0)),
            pl.BlockSpec((1, D1), lambda i: (0, 0)),
        ],
        out_specs=[
            pl.BlockSpec((blk, DP1), lambda i: (i, 0)),
            pl.BlockSpec((blk, 2), lambda i: (i, 0)),
        ],
        out_shape=[
            jax.ShapeDtypeStruct((NROWS, DP1), jnp.float32),
            jax.ShapeDtypeStruct((NROWS, 2), jnp.float32),
        ],
    )(x_pad, W1, att_src1.reshape(1, D1), att_dst1.reshape(1, D1))


def _edge_alpha_body(ea_ref, we1_ref, ae1_ref, we2_ref, ae2_ref,
                     a1_ref, a2_ref, sums_ref, cv_ref):
    i = pl.program_id(0)
    cs1 = jnp.sum(we1_ref[...] * ae1_ref[...], axis=1, keepdims=True)
    cs2 = jnp.sum(we2_ref[...] * ae2_ref[...], axis=1, keepdims=True)
    ea = ea_ref[...]
    a1_ref[...] = jnp.sum(ea * cs1, axis=0, keepdims=True)
    a2_ref[...] = jnp.sum(ea * cs2, axis=0, keepdims=True)

    @pl.when(i == 0)
    def _():
        sums_ref[...] = jnp.zeros_like(sums_ref)
        cv_ref[:, 0:1] = cs1
        cv_ref[:, 1:2] = cs2

    eb = ea.shape[1]
    sums_ref[...] += jnp.sum(ea.reshape(2, eb // 128, 128), axis=1)


def _edge_alpha(ea_t, We1, att_edge1, We2, att_edge2):
    eb = 12800
    return pl.pallas_call(
        _edge_alpha_body,
        grid=(E0 // eb,),
        in_specs=[
            pl.BlockSpec((2, eb), lambda i: (0, i)),
            pl.BlockSpec((2, D1), lambda i: (0, 0)),
            pl.BlockSpec((1, D1), lambda i: (0, 0)),
            pl.BlockSpec((2, D2), lambda i: (0, 0)),
            pl.BlockSpec((1, D2), lambda i: (0, 0)),
        ],
        out_specs=[
            pl.BlockSpec((1, eb), lambda i: (0, i)),
            pl.BlockSpec((1, eb), lambda i: (0, i)),
            pl.BlockSpec((2, 128), lambda i: (0, 0)),
            pl.BlockSpec((2, 2), lambda i: (0, 0)),
        ],
        out_shape=[
            jax.ShapeDtypeStruct((1, E0), jnp.float32),
            jax.ShapeDtypeStruct((1, E0), jnp.float32),
            jax.ShapeDtypeStruct((2, 128), jnp.float32),
            jax.ShapeDtypeStruct((2, 2), jnp.float32),
        ],
        compiler_params=pltpu.CompilerParams(
            dimension_semantics=("arbitrary",)),
    )(ea_t, We1, att_edge1.reshape(1, D1), We2, att_edge2.reshape(1, D2))


def _make_sc_pass(dpad):
    mesh = plsc.VectorSubcoreMesh(core_axis_name="c", subcore_axis_name="s")

    @functools.partial(
        pl.kernel,
        out_type=jax.ShapeDtypeStruct((NC * NROWS, dpad), jnp.float32),
        mesh=mesh,
        scratch_types=[
            pltpu.VMEM_SHARED((NROWS, dpad), jnp.float32),
            pltpu.VMEM((NROWS, 2), jnp.float32),
            pltpu.VMEM((CH,), jnp.int32),
            pltpu.VMEM((CH,), jnp.int32),
            pltpu.VMEM((CH,), jnp.float32),
            pltpu.VMEM((CH,), jnp.float32),
            pltpu.VMEM((CH, dpad), jnp.float32),
            pltpu.SemaphoreType.DMA,
        ],
    )
    def sc_pass(htab, anode, srcs, dsts, aes, zrows, out,
                acc_sh, an_v, src_v, dst_v, ae_v, ex_v, gbuf, sem):
        c = lax.axis_index("c")
        s = lax.axis_index("s")
        wid = c * NS + s
        row0 = s * RPS
        rem = RPS - 4 * CH

        pltpu.sync_copy(zrows, gbuf)
        for r in range(4):
            pltpu.sync_copy(gbuf, acc_sh.at[pl.ds(row0 + r * CH, CH)])
        pltpu.sync_copy(gbuf.at[pl.ds(0, rem)],
                        acc_sh.at[pl.ds(row0 + 4 * CH, rem)])
        pltpu.sync_copy(anode, an_v)
        plsc.subcore_barrier()

        zero16 = jnp.zeros((16,), jnp.int32)
        one16 = jnp.ones((16,), jnp.int32)

        def chunk(k, carry):
            base = wid * TE + k * CH
            pltpu.sync_copy(srcs.at[pl.ds(base, CH)], src_v)
            pltpu.sync_copy(dsts.at[pl.ds(base, CH)], dst_v)
            pltpu.sync_copy(aes.at[pl.ds(base, CH)], ae_v)
            pltpu.async_copy(htab.at[src_v], gbuf, sem).wait()
            for j in range(CH // 16):
                si = src_v[pl.ds(j * 16, 16)]
                di = dst_v[pl.ds(j * 16, 16)]
                av = plsc.load_gather(an_v, [si, zero16])
                bv = plsc.load_gather(an_v, [di, one16])
                a = av + bv + ae_v[pl.ds(j * 16, 16)]
                a = jnp.where(a > 0, a, a * NEG_SLOPE)
                ex_v[pl.ds(j * 16, 16)] = jnp.exp(a)

            def scale(e, cc):
                exb = lax.broadcast(ex_v[e], (16,))
                for j in range(dpad // 16):
                    gbuf[e, pl.ds(j * 16, 16)] = gbuf[e, pl.ds(j * 16, 16)] * exb
                return cc

            lax.fori_loop(0, CH, scale, 0)
            pltpu.sync_copy(gbuf, acc_sh.at[dst_v], add=True)
            return carry

        lax.fori_loop(0, NCHUNK, chunk, 0)
        plsc.subcore_barrier()

        out_base = c * NROWS + s * RPS
        for r in range(4):
            pltpu.sync_copy(acc_sh.at[pl.ds(row0 + r * CH, CH)], gbuf)
            pltpu.sync_copy(gbuf, out.at[pl.ds(out_base + r * CH, CH)])
        pltpu.sync_copy(acc_sh.at[pl.ds(row0 + 4 * CH, rem)],
                        gbuf.at[pl.ds(0, rem)])
        pltpu.sync_copy(gbuf.at[pl.ds(0, rem)],
                        out.at[pl.ds(out_base + 4 * CH, rem)])

    return sc_pass


_sc_pass_1 = _make_sc_pass(DP1)
_sc_pass_2 = _make_sc_pass(DP2)


def _combine1_body(acc_ref, b_ref, w_ref, as_ref, ad_ref, htab_ref, an_ref):
    i = pl.program_id(0)
    blk = htab_ref.shape[0]
    accs = acc_ref[0] + acc_ref[1]
    denom = accs[:, D1:D1 + 1]
    h1 = jnp.maximum(accs[:, :D1] / denom + b_ref[...], 0.0)
    rows = i * blk + lax.broadcasted_iota(jnp.int32, (blk, 1), 0)
    mask = rows < N
    h1 = jnp.where(mask, h1, 0.0)
    h2 = jnp.dot(h1, w_ref[...], preferred_element_type=jnp.float32)
    htab_ref[:, :D2] = h2
    htab_ref[:, D2:D2 + 1] = jnp.where(mask, 1.0, 0.0)
    htab_ref[:, D2 + 1:] = jnp.zeros((blk, DP2 - D2 - 1), jnp.float32)
    an_ref[:, 0:1] = jnp.sum(h2 * as_ref[...], axis=1, keepdims=True)
    an_ref[:, 1:2] = jnp.sum(h2 * ad_ref[...], axis=1, keepdims=True)


def _combine1(acc1, b1, W2, att_src2, att_dst2):
    blk = 2504
    acc1 = acc1.reshape(NC, NROWS, DP1)
    return pl.pallas_call(
        _combine1_body,
        grid=(NROWS // blk,),
        in_specs=[
            pl.BlockSpec((NC, blk, DP1), lambda i: (0, i, 0)),
            pl.BlockSpec((1, D1), lambda i: (0, 0)),
            pl.BlockSpec((D1, D2), lambda i: (0, 0)),
            pl.BlockSpec((1, D2), lambda i: (0, 0)),
            pl.BlockSpec((1, D2), lambda i: (0, 0)),
        ],
        out_specs=[
            pl.BlockSpec((blk, DP2), lambda i: (i, 0)),
            pl.BlockSpec((blk, 2), lambda i: (i, 0)),
        ],
        out_shape=[
            jax.ShapeDtypeStruct((NROWS, DP2), jnp.float32),
            jax.ShapeDtypeStruct((NROWS, 2), jnp.float32),
        ],
    )(acc1, b1.reshape(1, D1), W2, att_src2.reshape(1, D2),
      att_dst2.reshape(1, D2))


def _final_body(acc_ref, b_ref, o_ref):
    accs = acc_ref[0] + acc_ref[1]
    out = accs[:, :D2] / accs[:, D2:D2 + 1] + b_ref[...]
    m = jnp.max(out, axis=1, keepdims=True)
    lse = jnp.log(jnp.sum(jnp.exp(out - m), axis=1, keepdims=True)) + m
    o_ref[...] = out - lse


def _final(acc2, b2):
    blk = 5000
    acc2 = acc2.reshape(NC, NROWS, DP2)
    return pl.pallas_call(
        _final_body,
        grid=(N // blk,),
        in_specs=[
            pl.BlockSpec((NC, blk, DP2), lambda i: (0, i, 0)),
            pl.BlockSpec((1, D2), lambda i: (0, 0)),
        ],
        out_specs=pl.BlockSpec((blk, D2), lambda i: (i, 0)),
        out_shape=jax.ShapeDtypeStruct((N, D2), jnp.float32),
    )(acc2, b2.reshape(1, D2))


def kernel(x, edge_index, edge_attr, W1, att_src1, att_dst1, We1, att_edge1,
           b1, W2, att_src2, att_dst2, We2, att_edge2, b2):
    loop = jnp.arange(N, dtype=jnp.int32)
    padi = jnp.full((EPAD - E,), N, dtype=jnp.int32)
    src = jnp.concatenate([edge_index[0].astype(jnp.int32), loop, padi])
    dst = jnp.concatenate([edge_index[1].astype(jnp.int32), loop, padi])
    x_pad = jnp.concatenate(
        [x, jnp.zeros((NROWS - N, D1), jnp.float32)], axis=0)
    ea_t = edge_attr.T
    zrows1 = jnp.zeros((CH, DP1), jnp.float32)
    zrows2 = jnp.zeros((CH, DP2), jnp.float32)

    htab1, anode1 = _prep1(x_pad, W1, att_src1, att_dst1)
    ae1_real, ae2_real, ea_sums, cvec = _edge_alpha(
        ea_t, We1, att_edge1, We2, att_edge2)
    ea_sum = jnp.sum(ea_sums, axis=1)
    aself1 = (cvec[0, 0] * ea_sum[0] + cvec[1, 0] * ea_sum[1]) / E0
    aself2 = (cvec[0, 1] * ea_sum[0] + cvec[1, 1] * ea_sum[1]) / E0
    zpad = jnp.zeros((EPAD - E,), jnp.float32)
    ae1 = jnp.concatenate(
        [ae1_real.reshape(E0), jnp.full((N,), aself1, jnp.float32), zpad])
    ae2 = jnp.concatenate(
        [ae2_real.reshape(E0), jnp.full((N,), aself2, jnp.float32), zpad])

    acc1 = _sc_pass_1(htab1, anode1, src, dst, ae1, zrows1)
    htab2, anode2 = _combine1(acc1, b1, W2, att_src2, att_dst2)

    acc2 = _sc_pass_2(htab2, anode2, src, dst, ae2, zrows2)
    return _final(acc2, b2)


# speedup vs baseline: 24.0156x; 24.0156x over previous
import functools

import jax
import jax.numpy as jnp
from jax import lax
from jax.experimental import pallas as pl
from jax.experimental.pallas import tpu as pltpu
from jax.experimental.pallas import tpu_sc as plsc

N = 10000
NROWS = 10112
RPS = NROWS // 16
E0 = 320000
E = E0 + N
NC, NS = 2, 16
NW = NC * NS
CH = 128
TE = 10368
EPAD = TE * NW
NCHUNK = TE // CH
D1, DP1 = 128, 144
D2, DP2 = 64, 80
NEG_SLOPE = 0.2


def _prep1_body(x_ref, w_ref, as_ref, ad_ref, htab_ref, an_ref):
    i = pl.program_id(0)
    blk = htab_ref.shape[0]
    h = jnp.dot(x_ref[...], w_ref[...], preferred_element_type=jnp.float32)
    rows = i * blk + lax.broadcasted_iota(jnp.int32, (blk, 1), 0)
    mask = rows < N
    h = jnp.where(mask, h, 0.0)
    htab_ref[:, :D1] = h
    htab_ref[:, D1:D1 + 1] = jnp.where(mask, 1.0, 0.0)
    htab_ref[:, D1 + 1:] = jnp.zeros((blk, DP1 - D1 - 1), jnp.float32)
    an_ref[:, 0:1] = jnp.sum(h * as_ref[...], axis=1, keepdims=True)
    an_ref[:, 1:2] = jnp.sum(h * ad_ref[...], axis=1, keepdims=True)


def _prep1(x_pad, W1, att_src1, att_dst1):
    blk = 2528
    return pl.pallas_call(
        _prep1_body,
        grid=(NROWS // blk,),
        in_specs=[
            pl.BlockSpec((blk, D1), lambda i: (i, 0)),
            pl.BlockSpec((D1, D1), lambda i: (0, 0)),
            pl.BlockSpec((1, D1), lambda i: (0, 0)),
            pl.BlockSpec((1, D1), lambda i: (0, 0)),
        ],
        out_specs=[
            pl.BlockSpec((blk, DP1), lambda i: (i, 0)),
            pl.BlockSpec((blk, 2), lambda i: (i, 0)),
        ],
        out_shape=[
            jax.ShapeDtypeStruct((NROWS, DP1), jnp.float32),
            jax.ShapeDtypeStruct((NROWS, 2), jnp.float32),
        ],
    )(x_pad, W1, att_src1.reshape(1, D1), att_dst1.reshape(1, D1))


def _edge_alpha_body(ea_ref, we1_ref, ae1_ref, we2_ref, ae2_ref,
                     a1_ref, a2_ref, sums_ref, cv_ref):
    i = pl.program_id(0)
    cs1 = jnp.sum(we1_ref[...] * ae1_ref[...], axis=1, keepdims=True)
    cs2 = jnp.sum(we2_ref[...] * ae2_ref[...], axis=1, keepdims=True)
    ea = ea_ref[...]
    a1_ref[...] = jnp.sum(ea * cs1, axis=0, keepdims=True)
    a2_ref[...] = jnp.sum(ea * cs2, axis=0, keepdims=True)

    @pl.when(i == 0)
    def _():
        sums_ref[...] = jnp.zeros_like(sums_ref)
        cv_ref[:, 0:1] = cs1
        cv_ref[:, 1:2] = cs2

    eb = ea.shape[1]
    sums_ref[...] += jnp.sum(ea.reshape(2, eb // 128, 128), axis=1)


def _edge_alpha(ea_t, We1, att_edge1, We2, att_edge2):
    eb = 12800
    return pl.pallas_call(
        _edge_alpha_body,
        grid=(E0 // eb,),
        in_specs=[
            pl.BlockSpec((2, eb), lambda i: (0, i)),
            pl.BlockSpec((2, D1), lambda i: (0, 0)),
            pl.BlockSpec((1, D1), lambda i: (0, 0)),
            pl.BlockSpec((2, D2), lambda i: (0, 0)),
            pl.BlockSpec((1, D2), lambda i: (0, 0)),
        ],
        out_specs=[
            pl.BlockSpec((1, eb), lambda i: (0, i)),
            pl.BlockSpec((1, eb), lambda i: (0, i)),
            pl.BlockSpec((2, 128), lambda i: (0, 0)),
            pl.BlockSpec((2, 2), lambda i: (0, 0)),
        ],
        out_shape=[
            jax.ShapeDtypeStruct((1, E0), jnp.float32),
            jax.ShapeDtypeStruct((1, E0), jnp.float32),
            jax.ShapeDtypeStruct((2, 128), jnp.float32),
            jax.ShapeDtypeStruct((2, 2), jnp.float32),
        ],
        compiler_params=pltpu.CompilerParams(
            dimension_semantics=("arbitrary",)),
    )(ea_t, We1, att_edge1.reshape(1, D1), We2, att_edge2.reshape(1, D2))


def _make_sc_pass(dpad):
    mesh = plsc.VectorSubcoreMesh(core_axis_name="c", subcore_axis_name="s",
                                  num_cores=NC, num_subcores=NS)

    @functools.partial(
        pl.kernel,
        out_type=jax.ShapeDtypeStruct((NC * NROWS, dpad), jnp.float32),
        mesh=mesh,
        scratch_types=[
            pltpu.VMEM_SHARED((NROWS, dpad), jnp.float32),
            pltpu.VMEM((2 * NROWS,), jnp.float32),
            pltpu.VMEM((CH,), jnp.int32),
            pltpu.VMEM((CH,), jnp.int32),
            pltpu.VMEM((CH,), jnp.float32),
            pltpu.VMEM((CH,), jnp.float32),
            pltpu.VMEM((CH, dpad), jnp.float32),
            pltpu.SemaphoreType.DMA,
        ],
        compiler_params=pltpu.CompilerParams(use_tc_tiling_on_sc=False,
                                             needs_layout_passes=False),
    )
    def sc_pass(htab, anode, srcs, dsts, aes, zrows, out,
                acc_sh, an_v, src_v, dst_v, ae_v, ex_v, gbuf, sem):
        c = lax.axis_index("c")
        s = lax.axis_index("s")
        wid = c * NS + s
        row0 = s * RPS
        rem = RPS - 4 * CH

        pltpu.sync_copy(zrows, gbuf)
        for r in range(4):
            pltpu.sync_copy(gbuf, acc_sh.at[pl.ds(row0 + r * CH, CH)])
        pltpu.sync_copy(gbuf.at[pl.ds(0, rem)],
                        acc_sh.at[pl.ds(row0 + 4 * CH, rem)])
        pltpu.sync_copy(anode, an_v)
        plsc.subcore_barrier()

        def chunk(k, carry):
            base = wid * TE + k * CH
            pltpu.sync_copy(srcs.at[pl.ds(base, CH)], src_v)
            pltpu.sync_copy(dsts.at[pl.ds(base, CH)], dst_v)
            pltpu.sync_copy(aes.at[pl.ds(base, CH)], ae_v)
            pltpu.async_copy(htab.at[src_v], gbuf, sem).wait()
            for j in range(CH // 16):
                si = src_v[pl.ds(j * 16, 16)]
                di = dst_v[pl.ds(j * 16, 16)]
                av = plsc.load_gather(an_v, [si * 2])
                bv = plsc.load_gather(an_v, [di * 2 + 1])
                a = av + bv + ae_v[pl.ds(j * 16, 16)]
                a = jnp.where(a > 0, a, a * NEG_SLOPE)
                ex_v[pl.ds(j * 16, 16)] = jnp.exp(a)

            def scale(g, cc):
                exv = ex_v[pl.ds(g * 16, 16)]
                e0 = g * 16
                for l in range(16):
                    exb = lax.broadcast(exv[l], (16,))
                    for j in range(dpad // 16):
                        gbuf[e0 + l, pl.ds(j * 16, 16)] = (
                            gbuf[e0 + l, pl.ds(j * 16, 16)] * exb)
                return cc

            lax.fori_loop(0, CH // 16, scale, 0)
            pltpu.sync_copy(gbuf, acc_sh.at[dst_v], add=True)
            return carry

        lax.fori_loop(0, NCHUNK, chunk, 0)
        plsc.subcore_barrier()

        out_base = c * NROWS + s * RPS
        for r in range(4):
            pltpu.sync_copy(acc_sh.at[pl.ds(row0 + r * CH, CH)], gbuf)
            pltpu.sync_copy(gbuf, out.at[pl.ds(out_base + r * CH, CH)])
        pltpu.sync_copy(acc_sh.at[pl.ds(row0 + 4 * CH, rem)],
                        gbuf.at[pl.ds(0, rem)])
        pltpu.sync_copy(gbuf.at[pl.ds(0, rem)],
                        out.at[pl.ds(out_base + 4 * CH, rem)])

    return sc_pass


_sc_pass_1 = _make_sc_pass(DP1)
_sc_pass_2 = _make_sc_pass(DP2)


def _combine1_body(acc_ref, b_ref, w_ref, as_ref, ad_ref, htab_ref, an_ref):
    i = pl.program_id(0)
    blk = htab_ref.shape[0]
    accs = acc_ref[0] + acc_ref[1]
    denom = accs[:, D1:D1 + 1]
    h1 = jnp.maximum(accs[:, :D1] / denom + b_ref[...], 0.0)
    rows = i * blk + lax.broadcasted_iota(jnp.int32, (blk, 1), 0)
    mask = rows < N
    h1 = jnp.where(mask, h1, 0.0)
    h2 = jnp.dot(h1, w_ref[...], preferred_element_type=jnp.float32)
    htab_ref[:, :D2] = h2
    htab_ref[:, D2:D2 + 1] = jnp.where(mask, 1.0, 0.0)
    htab_ref[:, D2 + 1:] = jnp.zeros((blk, DP2 - D2 - 1), jnp.float32)
    an_ref[:, 0:1] = jnp.sum(h2 * as_ref[...], axis=1, keepdims=True)
    an_ref[:, 1:2] = jnp.sum(h2 * ad_ref[...], axis=1, keepdims=True)


def _combine1(acc1, b1, W2, att_src2, att_dst2):
    blk = 2528
    acc1 = acc1.reshape(NC, NROWS, DP1)
    return pl.pallas_call(
        _combine1_body,
        grid=(NROWS // blk,),
        in_specs=[
            pl.BlockSpec((NC, blk, DP1), lambda i: (0, i, 0)),
            pl.BlockSpec((1, D1), lambda i: (0, 0)),
            pl.BlockSpec((D1, D2), lambda i: (0, 0)),
            pl.BlockSpec((1, D2), lambda i: (0, 0)),
            pl.BlockSpec((1, D2), lambda i: (0, 0)),
        ],
        out_specs=[
            pl.BlockSpec((blk, DP2), lambda i: (i, 0)),
            pl.BlockSpec((blk, 2), lambda i: (i, 0)),
        ],
        out_shape=[
            jax.ShapeDtypeStruct((NROWS, DP2), jnp.float32),
            jax.ShapeDtypeStruct((NROWS, 2), jnp.float32),
        ],
    )(acc1, b1.reshape(1, D1), W2, att_src2.reshape(1, D2),
      att_dst2.reshape(1, D2))


def _final_body(acc_ref, b_ref, o_ref):
    accs = acc_ref[0] + acc_ref[1]
    out = accs[:, :D2] / accs[:, D2:D2 + 1] + b_ref[...]
    m = jnp.max(out, axis=1, keepdims=True)
    lse = jnp.log(jnp.sum(jnp.exp(out - m), axis=1, keepdims=True)) + m
    o_ref[...] = out - lse


def _final(acc2, b2):
    blk = 5000
    acc2 = acc2.reshape(NC, NROWS, DP2)
    return pl.pallas_call(
        _final_body,
        grid=(N // blk,),
        in_specs=[
            pl.BlockSpec((NC, blk, DP2), lambda i: (0, i, 0)),
            pl.BlockSpec((1, D2), lambda i: (0, 0)),
        ],
        out_specs=pl.BlockSpec((blk, D2), lambda i: (i, 0)),
        out_shape=jax.ShapeDtypeStruct((N, D2), jnp.float32),
    )(acc2, b2.reshape(1, D2))


def kernel(x, edge_index, edge_attr, W1, att_src1, att_dst1, We1, att_edge1,
           b1, W2, att_src2, att_dst2, We2, att_edge2, b2):
    loop = jnp.arange(N, dtype=jnp.int32)
    padi = jnp.full((EPAD - E,), N, dtype=jnp.int32)
    src = jnp.concatenate([edge_index[0].astype(jnp.int32), loop, padi])
    dst = jnp.concatenate([edge_index[1].astype(jnp.int32), loop, padi])
    x_pad = jnp.concatenate(
        [x, jnp.zeros((NROWS - N, D1), jnp.float32)], axis=0)
    ea_t = edge_attr.T
    zrows1 = jnp.zeros((CH, DP1), jnp.float32)
    zrows2 = jnp.zeros((CH, DP2), jnp.float32)

    htab1, anode1 = _prep1(x_pad, W1, att_src1, att_dst1)
    ae1_real, ae2_real, ea_sums, cvec = _edge_alpha(
        ea_t, We1, att_edge1, We2, att_edge2)
    ea_sum = jnp.sum(ea_sums, axis=1)
    aself1 = (cvec[0, 0] * ea_sum[0] + cvec[1, 0] * ea_sum[1]) / E0
    aself2 = (cvec[0, 1] * ea_sum[0] + cvec[1, 1] * ea_sum[1]) / E0
    zpad = jnp.zeros((EPAD - E,), jnp.float32)
    ae1 = jnp.concatenate(
        [ae1_real.reshape(E0), jnp.full((N,), aself1, jnp.float32), zpad])
    ae2 = jnp.concatenate(
        [ae2_real.reshape(E0), jnp.full((N,), aself2, jnp.float32), zpad])

    acc1 = _sc_pass_1(htab1, anode1.reshape(2 * NROWS), src, dst, ae1, zrows1)
    htab2, anode2 = _combine1(acc1, b1, W2, att_src2, att_dst2)

    acc2 = _sc_pass_2(htab2, anode2.reshape(2 * NROWS), src, dst, ae2, zrows2)
    return _final(acc2, b2)
